# baseline (speedup 1.0000x reference)
import functools
import jax
import jax.numpy as jnp
from jax import lax
from jax.experimental import pallas as pl
from jax.experimental.pallas import tpu as pltpu
from jax.experimental.pallas import tpu_sc as plsc

NC, NS = 2, 16
NW = NC * NS
R_SC = 16


def _make_sc(batch, seq, dim):
    rows_per_w = seq // NW
    nchunk = rows_per_w // R_SC
    nvec = dim // 16
    mesh = plsc.VectorSubcoreMesh(core_axis_name="c", subcore_axis_name="s")

    @functools.partial(
        pl.kernel,
        mesh=mesh,
        out_type=jax.ShapeDtypeStruct((batch, seq, dim), jnp.float32),
        scratch_types=[
            pltpu.VMEM((16,), jnp.float32),
            pltpu.VMEM((R_SC, dim), jnp.float32),
            pltpu.VMEM((R_SC, dim), jnp.float32),
            pltpu.VMEM((R_SC, dim), jnp.float32),
        ],
    )
    def sc_fn(in_hbm, pe_hbm, scale_hbm, out_hbm, scale_v, pe_v, in_v, out_v):
        wid = lax.axis_index("s") * NC + lax.axis_index("c")
        base = wid * rows_per_w
        pltpu.sync_copy(scale_hbm, scale_v)
        s = scale_v[...]

        def chunk_body(c, _):
            r0 = base + c * R_SC
            pltpu.sync_copy(pe_hbm.at[pl.ds(r0, R_SC), :], pe_v)

            def batch_body(b, _):
                pltpu.sync_copy(in_hbm.at[b, pl.ds(r0, R_SC), :], in_v)

                def row_body(r, _):
                    def vec_body(j, _):
                        sl = pl.ds(j * 16, 16)
                        out_v[r, sl] = in_v[r, sl] + pe_v[r, sl] * s
                        return 0
                    lax.fori_loop(0, nvec, vec_body, 0)
                    return 0

                lax.fori_loop(0, R_SC, row_body, 0)
                pltpu.sync_copy(out_v, out_hbm.at[b, pl.ds(r0, R_SC), :])
                return 0

            lax.fori_loop(0, batch, batch_body, 0)
            return 0

        lax.fori_loop(0, nchunk, chunk_body, 0)

    return sc_fn


def kernel(input, pe, scale_param):
    batch, seq, dim = input.shape
    scale16 = jnp.broadcast_to(scale_param, (16,))
    return _make_sc(batch, seq, dim)(input, pe[:seq], scale16)


# speedup vs baseline: 3.6101x; 3.6101x over previous
import functools
import jax
import jax.numpy as jnp
from jax import lax
from jax.experimental import pallas as pl
from jax.experimental.pallas import tpu as pltpu
from jax.experimental.pallas import tpu_sc as plsc

R = 256
NBUF = 4

NC, NS = 2, 16
NW = NC * NS
R_SC = 16
SC_ROWS = 1024


def _tc_body(scale_ref, in_hbm, pe_hbm, out_hbm,
             in_v, pe_v, out_v, in_sem, pe_sem, out_sem, *, tc_nchunk):
    s = scale_ref[0]

    def in_copy(j, slot):
        return pltpu.make_async_copy(
            in_hbm.at[:, pl.ds(j * R, R), :], in_v.at[slot], in_sem.at[slot])

    def pe_copy(j, slot):
        return pltpu.make_async_copy(
            pe_hbm.at[pl.ds(j * R, R), :], pe_v.at[slot], pe_sem.at[slot])

    def out_copy(j, slot):
        return pltpu.make_async_copy(
            out_v.at[slot], out_hbm.at[:, pl.ds(j * R, R), :], out_sem.at[slot])

    for k in range(NBUF):
        in_copy(k, k).start()
        pe_copy(k, k).start()

    def body(j, carry):
        slot = lax.rem(j, NBUF)
        in_copy(j, slot).wait()
        pe_copy(j, slot).wait()

        @pl.when(j >= NBUF)
        def _():
            out_copy(j - NBUF, slot).wait()

        out_v[slot] = in_v[slot] + s * pe_v[slot][None, :, :]
        out_copy(j, slot).start()

        nxt = j + NBUF

        @pl.when(nxt < tc_nchunk)
        def _():
            in_copy(nxt, slot).start()
            pe_copy(nxt, slot).start()

        return carry

    lax.fori_loop(0, tc_nchunk, body, 0)

    for k in range(NBUF):
        out_copy(tc_nchunk - NBUF + k, k).wait()


def _make_sc(batch, dim, tc_rows):
    rows_per_w = SC_ROWS // NW
    sc_nchunk = rows_per_w // R_SC
    nvec = dim // 16
    mesh = plsc.VectorSubcoreMesh(core_axis_name="c", subcore_axis_name="s")

    @functools.partial(
        pl.kernel,
        mesh=mesh,
        out_type=jax.ShapeDtypeStruct((batch, SC_ROWS, dim), jnp.float32),
        scratch_types=[
            pltpu.VMEM((16,), jnp.float32),
            pltpu.VMEM((R_SC, dim), jnp.float32),
            pltpu.VMEM((R_SC, dim), jnp.float32),
            pltpu.VMEM((R_SC, dim), jnp.float32),
        ],
    )
    def sc_fn(in_hbm, pe_hbm, scale_hbm, out_hbm, scale_v, pe_v, in_v, out_v):
        wid = lax.axis_index("s") * NC + lax.axis_index("c")
        base = wid * rows_per_w
        pltpu.sync_copy(scale_hbm, scale_v)
        s = scale_v[...]

        def chunk_body(c, _):
            r0 = base + c * R_SC

            pltpu.sync_copy(pe_hbm.at[pl.ds(tc_rows + r0, R_SC), :], pe_v)

            def batch_body(b, _):
                pltpu.sync_copy(
                    in_hbm.at[b, pl.ds(tc_rows + r0, R_SC), :], in_v)

                def row_body(r, _):
                    def vec_body(j, _):
                        sl = pl.ds(j * 16, 16)
                        out_v[r, sl] = in_v[r, sl] + pe_v[r, sl] * s
                        return 0
                    lax.fori_loop(0, nvec, vec_body, 0)
                    return 0

                lax.fori_loop(0, R_SC, row_body, 0)
                pltpu.sync_copy(out_v, out_hbm.at[b, pl.ds(r0, R_SC), :])
                return 0

            lax.fori_loop(0, batch, batch_body, 0)
            return 0

        lax.fori_loop(0, sc_nchunk, chunk_body, 0)

    return sc_fn


def kernel(input, pe, scale_param):
    batch, seq, dim = input.shape
    tc_rows = seq - SC_ROWS
    tc_nchunk = tc_rows // R

    tc_out = pl.pallas_call(
        functools.partial(_tc_body, tc_nchunk=tc_nchunk),
        in_specs=[
            pl.BlockSpec(memory_space=pltpu.SMEM),
            pl.BlockSpec(memory_space=pl.ANY),
            pl.BlockSpec(memory_space=pl.ANY),
        ],
        out_specs=pl.BlockSpec(memory_space=pl.ANY),
        out_shape=jax.ShapeDtypeStruct((batch, seq, dim), input.dtype),
        scratch_shapes=[
            pltpu.VMEM((NBUF, batch, R, dim), input.dtype),
            pltpu.VMEM((NBUF, R, dim), pe.dtype),
            pltpu.VMEM((NBUF, batch, R, dim), input.dtype),
            pltpu.SemaphoreType.DMA((NBUF,)),
            pltpu.SemaphoreType.DMA((NBUF,)),
            pltpu.SemaphoreType.DMA((NBUF,)),
        ],
    )(scale_param, input, pe[:seq])

    scale16 = jnp.broadcast_to(scale_param, (16,))
    sc_out = _make_sc(batch, dim, tc_rows)(input, pe[:seq], scale16)

    return lax.dynamic_update_slice(tc_out, sc_out, (0, tc_rows, 0))
